# baseline (speedup 1.0000x reference)
import functools

import jax
import jax.numpy as jnp
from jax import lax
from jax.experimental import pallas as pl
from jax.experimental.pallas import tpu as pltpu
from jax.experimental.pallas import tpu_sc as plsc

N = 10000
D = 128
E_PAIR = 300000
E_LAB = 10000
B = 2 * E_PAIR + E_LAB
CHUNK = 128
NW = 32
CPC = 75
H_ROWS = NW * CPC * CHUNK
B_PAD = 2 * H_ROWS
ACC_ROWS = 10112
RPT = ACC_ROWS // 16
BLK = 600
NBLK_H = (H_ROWS // 2) // BLK
BPR = E_PAIR // 2 // BLK
UBLK = 1000

_SC_MESH = plsc.VectorSubcoreMesh(core_axis_name="c", subcore_axis_name="s")


def _worker_id():
    return lax.axis_index("s") * 2 + lax.axis_index("c")


@functools.partial(
    pl.kernel,
    out_type=jax.ShapeDtypeStruct((H_ROWS, D), jnp.float32),
    mesh=_SC_MESH,
    scratch_types=[
        pltpu.VMEM((CPC, CHUNK), jnp.int32),
        [pltpu.VMEM((CHUNK, D), jnp.float32) for _ in range(2)],
        pltpu.SemaphoreType.DMA,
        [pltpu.SemaphoreType.DMA for _ in range(2)],
    ],
)
def _gather_k(tbl_hbm, idx_hbm, out_hbm, idx_v, bufs, semg, semw):
    w = _worker_id()
    c0 = w * CPC
    pltpu.sync_copy(idx_hbm.at[w], idx_v)

    def fire_w(t, j):
        pltpu.async_copy(bufs[j], out_hbm.at[pl.ds((c0 + t) * CHUNK, CHUNK)],
                         semw[j])

    def drain_w(t, j):
        pltpu.make_async_copy(bufs[j], out_hbm.at[pl.ds((c0 + t) * CHUNK, CHUNK)],
                              semw[j]).wait()

    def fire_g(t, j, sem):
        pltpu.async_copy(tbl_hbm.at[idx_v.at[t]], bufs[j], sem)

    def drain_g(t, j, sem):
        pltpu.make_async_copy(tbl_hbm.at[idx_v.at[t]], bufs[j], sem).wait()

    fire_g(0, 0, semg)
    fire_g(1, 1, semw[0])

    def body(i, carry):
        t0 = 2 * i
        t1 = t0 + 1
        drain_g(t0, 0, semg)
        pltpu.sync_copy(bufs[0], out_hbm.at[pl.ds((c0 + t0) * CHUNK, CHUNK)])

        @pl.when(t0 + 2 < CPC)
        def _():
            fire_g(t0 + 2, 0, semg)

        drain_g(t1, 1, semw[0])
        pltpu.sync_copy(bufs[1], out_hbm.at[pl.ds((c0 + t1) * CHUNK, CHUNK)])

        @pl.when(t1 + 2 < CPC)
        def _():
            fire_g(t1 + 2, 1, semw[0])

        return carry

    lax.fori_loop(0, CPC // 2, body, 0)
    drain_g(CPC - 1, 0, semg)
    pltpu.sync_copy(bufs[0], out_hbm.at[pl.ds((c0 + CPC - 1) * CHUNK, CHUNK)])


@functools.partial(
    pl.kernel,
    out_type=jax.ShapeDtypeStruct((2, ACC_ROWS, D), jnp.float32),
    mesh=_SC_MESH,
    scratch_types=[
        pltpu.VMEM((CPC, CHUNK), jnp.int32),
        [pltpu.VMEM((CHUNK, D), jnp.float32) for _ in range(2)],
        pltpu.VMEM_SHARED((ACC_ROWS, D), jnp.float32),
        [pltpu.SemaphoreType.DMA for _ in range(2)],
        [pltpu.SemaphoreType.DMA for _ in range(2)],
    ],
)
def _scatter_k(y_hbm, idx_hbm, zeros_hbm, out_hbm, idx_v, bufs, acc, semr, sema):
    c = lax.axis_index("c")
    s = lax.axis_index("s")
    w = s * 2 + c
    c0 = w * CPC
    pltpu.sync_copy(zeros_hbm.at[pl.ds(s * RPT, RPT)], acc.at[pl.ds(s * RPT, RPT)])
    pltpu.sync_copy(idx_hbm.at[w], idx_v)
    plsc.subcore_barrier()

    def fire_r(t, j):
        pltpu.async_copy(y_hbm.at[pl.ds((c0 + t) * CHUNK, CHUNK)], bufs[j], semr[j])

    def drain_r(t, j):
        pltpu.make_async_copy(y_hbm.at[pl.ds((c0 + t) * CHUNK, CHUNK)], bufs[j],
                              semr[j]).wait()

    def fire_a(t, j):
        pltpu.async_copy(bufs[j], acc.at[idx_v.at[t]], sema[j], add=True)

    def drain_a(t, j):
        pltpu.make_async_copy(bufs[j], acc.at[idx_v.at[t]], sema[j]).wait()

    fire_r(0, 0)
    fire_r(1, 1)

    def body(i, carry):
        t0 = 2 * i
        t1 = t0 + 1
        drain_r(t0, 0)
        fire_a(t0, 0)
        drain_r(t1, 1)
        fire_a(t1, 1)
        drain_a(t0, 0)

        @pl.when(t0 + 2 < CPC)
        def _():
            fire_r(t0 + 2, 0)

        drain_a(t1, 1)

        @pl.when(t1 + 2 < CPC)
        def _():
            fire_r(t1 + 2, 1)

        return carry

    lax.fori_loop(0, CPC // 2, body, 0)
    drain_r(CPC - 1, 0)
    fire_a(CPC - 1, 0)
    drain_a(CPC - 1, 0)
    plsc.subcore_barrier()
    pltpu.sync_copy(acc.at[pl.ds(s * RPT, RPT)], out_hbm.at[c, pl.ds(s * RPT, RPT)])


def _mish(z):
    sp = jnp.maximum(z, 0.0) + jnp.log1p(jnp.exp(-jnp.abs(z)))
    return z * jnp.tanh(sp)


def _dot(a, b):
    return jnp.dot(a, b, preferred_element_type=jnp.float32,
                   precision=lax.Precision.DEFAULT)


def _mlp_body(x_ref, wi_ref, bi_ref, wo_ref, bo_ref, y_ref):
    x = x_ref[...]
    z = _mish(_dot(x, wi_ref[0]) + bi_ref[0])
    y_ref[...] = x + _dot(z, wo_ref[0]) + bo_ref[0]


def _rel_mlp(x2, wi_s, bi_s, wo_s, bo_s, off):
    wsel = lambda i: (jnp.minimum((i + off) // BPR, 2), 0, 0)
    return pl.pallas_call(
        _mlp_body,
        grid=(NBLK_H,),
        in_specs=[
            pl.BlockSpec((BLK, 2 * D), lambda i: (i, 0)),
            pl.BlockSpec((1, 2 * D, 2 * D), wsel),
            pl.BlockSpec((1, 1, 2 * D), wsel),
            pl.BlockSpec((1, 2 * D, 2 * D), wsel),
            pl.BlockSpec((1, 1, 2 * D), wsel),
        ],
        out_specs=pl.BlockSpec((BLK, 2 * D), lambda i: (i, 0)),
        out_shape=jax.ShapeDtypeStruct((H_ROWS // 2, 2 * D), jnp.float32),
    )(x2, wi_s, bi_s, wo_s, bo_s)


def _upd_body(pa0, pa1, pb0, pb1, h_ref, wt_ref, wb_ref, bi_ref, wo_ref, bo_ref,
              o_ref):
    sm = pa0[0] + pa1[0] + pb0[0] + pb1[0]
    h = h_ref[...]
    z = _mish(_dot(sm, wt_ref[...]) + _dot(h, wb_ref[...]) + bi_ref[...])
    o_ref[...] = h + _dot(z, wo_ref[...]) + bo_ref[...]


def _update(pa, pb, h, wt, wb, bi, wo, bo):
    pspec = lambda ci: pl.BlockSpec((1, UBLK, D), lambda i, ci=ci: (ci, i, 0))
    return pl.pallas_call(
        _upd_body,
        grid=(N // UBLK,),
        in_specs=[
            pspec(0), pspec(1), pspec(0), pspec(1),
            pl.BlockSpec((UBLK, D), lambda i: (i, 0)),
            pl.BlockSpec((D, 2 * D), lambda i: (0, 0)),
            pl.BlockSpec((D, 2 * D), lambda i: (0, 0)),
            pl.BlockSpec((1, 2 * D), lambda i: (0, 0)),
            pl.BlockSpec((2 * D, D), lambda i: (0, 0)),
            pl.BlockSpec((1, D), lambda i: (0, 0)),
        ],
        out_specs=pl.BlockSpec((UBLK, D), lambda i: (i, 0)),
        out_shape=jax.ShapeDtypeStruct((N, D), jnp.float32),
    )(pa, pa, pb, pb, h, wt, wb, bi, wo, bo)


def _blockdiag(w):
    z = jnp.zeros((2 * D, 2 * D), jnp.float32)
    return z.at[:D, :D].set(w).at[D:, D:].set(w)


def kernel(node_embeddings, atoms_adj, atoms_goal_adj, atoms_label,
           Wi_adj, bi_adj, Wo_adj, bo_adj,
           Wi_goal_adj, bi_goal_adj, Wo_goal_adj, bo_goal_adj,
           Wi_label, bi_label, Wo_label, bo_label,
           Wi_upd, bi_upd, Wo_upd, bo_upd):
    idx = jnp.concatenate([atoms_adj, atoms_goal_adj, atoms_label]).astype(jnp.int32)
    gidx = jnp.concatenate(
        [idx, jnp.zeros((B_PAD - B,), jnp.int32)]).reshape(2, NW, CPC, CHUNK)
    sidx = jnp.concatenate(
        [idx, jnp.full((B_PAD - B,), N, jnp.int32)]).reshape(2, NW, CPC, CHUNK)
    zeros_acc = jnp.zeros((ACC_ROWS, D), jnp.float32)

    wi_s = jnp.stack([Wi_adj, Wi_goal_adj, _blockdiag(Wi_label)])
    wo_s = jnp.stack([Wo_adj, Wo_goal_adj, _blockdiag(Wo_label)])
    bi_s = jnp.stack([bi_adj, bi_goal_adj,
                      jnp.concatenate([bi_label, bi_label])]).reshape(3, 1, 2 * D)
    bo_s = jnp.stack([bo_adj, bo_goal_adj,
                      jnp.concatenate([bo_label, bo_label])]).reshape(3, 1, 2 * D)

    wt = Wi_upd[:D]
    wb = Wi_upd[D:]
    bi_u = bi_upd.reshape(1, 2 * D)
    bo_u = bo_upd.reshape(1, D)

    h = node_embeddings
    for _ in range(2):
        xa = _gather_k(h, gidx[0])
        ya = _rel_mlp(xa.reshape(H_ROWS // 2, 2 * D), wi_s, bi_s, wo_s, bo_s, 0)
        xb = _gather_k(h, gidx[1])
        yb = _rel_mlp(xb.reshape(H_ROWS // 2, 2 * D), wi_s, bi_s, wo_s, bo_s,
                      NBLK_H)
        pa = _scatter_k(ya.reshape(H_ROWS, D), sidx[0], zeros_acc)
        pb = _scatter_k(yb.reshape(H_ROWS, D), sidx[1], zeros_acc)
        h = _update(pa, pb, h, wt, wb, bi_u, Wo_upd, bo_u)
    return h


# speedup vs baseline: 1.0195x; 1.0195x over previous
import functools

import jax
import jax.numpy as jnp
from jax import lax
from jax.experimental import pallas as pl
from jax.experimental.pallas import tpu as pltpu
from jax.experimental.pallas import tpu_sc as plsc

N = 10000
D = 128
E_PAIR = 300000
E_LAB = 10000
B = 2 * E_PAIR + E_LAB
CHUNK = 128
NW = 32
CPC = 75
H_ROWS = NW * CPC * CHUNK
B_PAD = 2 * H_ROWS
ACC_ROWS = 10112
RPT = ACC_ROWS // 16
BLK = 600
NBLK_H = (H_ROWS // 2) // BLK
BPR = E_PAIR // 2 // BLK
UBLK = 1000

_SC_MESH = plsc.VectorSubcoreMesh(core_axis_name="c", subcore_axis_name="s")


def _worker_id():
    return lax.axis_index("s") * 2 + lax.axis_index("c")


@functools.partial(
    pl.kernel,
    out_type=jax.ShapeDtypeStruct((H_ROWS, D), jnp.float32),
    mesh=_SC_MESH,
    scratch_types=[
        pltpu.VMEM((CPC, CHUNK), jnp.int32),
        [pltpu.VMEM((CHUNK, D), jnp.float32) for _ in range(2)],
        pltpu.SemaphoreType.DMA,
        [pltpu.SemaphoreType.DMA for _ in range(2)],
    ],
)
def _gather_k(tbl_hbm, idx_hbm, out_hbm, idx_v, bufs, semg, semw):
    w = _worker_id()
    c0 = w * CPC
    pltpu.sync_copy(idx_hbm.at[w], idx_v)

    def fire_w(t, j):
        pltpu.async_copy(bufs[j], out_hbm.at[pl.ds((c0 + t) * CHUNK, CHUNK)],
                         semw[j])

    def drain_w(t, j):
        pltpu.make_async_copy(bufs[j], out_hbm.at[pl.ds((c0 + t) * CHUNK, CHUNK)],
                              semw[j]).wait()

    def fire_g(t, j, sem):
        pltpu.async_copy(tbl_hbm.at[idx_v.at[t]], bufs[j], sem)

    def drain_g(t, j, sem):
        pltpu.make_async_copy(tbl_hbm.at[idx_v.at[t]], bufs[j], sem).wait()

    fire_g(0, 0, semg)
    fire_g(1, 1, semw[0])

    def body(i, carry):
        t0 = 2 * i
        t1 = t0 + 1
        drain_g(t0, 0, semg)
        pltpu.sync_copy(bufs[0], out_hbm.at[pl.ds((c0 + t0) * CHUNK, CHUNK)])

        @pl.when(t0 + 2 < CPC)
        def _():
            fire_g(t0 + 2, 0, semg)

        drain_g(t1, 1, semw[0])
        pltpu.sync_copy(bufs[1], out_hbm.at[pl.ds((c0 + t1) * CHUNK, CHUNK)])

        @pl.when(t1 + 2 < CPC)
        def _():
            fire_g(t1 + 2, 1, semw[0])

        return carry

    lax.fori_loop(0, CPC // 2, body, 0)
    drain_g(CPC - 1, 0, semg)
    pltpu.sync_copy(bufs[0], out_hbm.at[pl.ds((c0 + CPC - 1) * CHUNK, CHUNK)])


@functools.partial(
    pl.kernel,
    out_type=jax.ShapeDtypeStruct((2, ACC_ROWS, D), jnp.float32),
    mesh=_SC_MESH,
    scratch_types=[
        pltpu.VMEM((CPC, CHUNK), jnp.int32),
        [pltpu.VMEM((CHUNK, D), jnp.float32) for _ in range(2)],
        pltpu.VMEM_SHARED((ACC_ROWS, D), jnp.float32),
        [pltpu.SemaphoreType.DMA for _ in range(2)],
    ],
)
def _scatter_k(y_hbm, idx_hbm, zeros_hbm, out_hbm, idx_v, bufs, acc, semr):
    c = lax.axis_index("c")
    s = lax.axis_index("s")
    w = s * 2 + c
    c0 = w * CPC
    pltpu.sync_copy(zeros_hbm.at[pl.ds(s * RPT, RPT)], acc.at[pl.ds(s * RPT, RPT)])
    pltpu.sync_copy(idx_hbm.at[w], idx_v)
    plsc.subcore_barrier()

    def fire_r(t, j):
        pltpu.async_copy(y_hbm.at[pl.ds((c0 + t) * CHUNK, CHUNK)], bufs[j], semr[j])

    def drain_r(t, j):
        pltpu.make_async_copy(y_hbm.at[pl.ds((c0 + t) * CHUNK, CHUNK)], bufs[j],
                              semr[j]).wait()

    fire_r(0, 0)
    fire_r(1, 1)

    def body(i, carry):
        t0 = 2 * i
        t1 = t0 + 1
        drain_r(t0, 0)
        pltpu.sync_copy(bufs[0], acc.at[idx_v.at[t0]], add=True)

        @pl.when(t0 + 2 < CPC)
        def _():
            fire_r(t0 + 2, 0)

        drain_r(t1, 1)
        pltpu.sync_copy(bufs[1], acc.at[idx_v.at[t1]], add=True)

        @pl.when(t1 + 2 < CPC)
        def _():
            fire_r(t1 + 2, 1)

        return carry

    lax.fori_loop(0, CPC // 2, body, 0)
    drain_r(CPC - 1, 0)
    pltpu.sync_copy(bufs[0], acc.at[idx_v.at[CPC - 1]], add=True)
    plsc.subcore_barrier()
    pltpu.sync_copy(acc.at[pl.ds(s * RPT, RPT)], out_hbm.at[c, pl.ds(s * RPT, RPT)])


def _mish(z):
    sp = jnp.maximum(z, 0.0) + jnp.log1p(jnp.exp(-jnp.abs(z)))
    return z * jnp.tanh(sp)


def _dot(a, b):
    return jnp.dot(a, b, preferred_element_type=jnp.float32,
                   precision=lax.Precision.DEFAULT)


def _mlp_body(x_ref, wi_ref, bi_ref, wo_ref, bo_ref, y_ref):
    x = x_ref[...]
    z = _mish(_dot(x, wi_ref[0]) + bi_ref[0])
    y_ref[...] = x + _dot(z, wo_ref[0]) + bo_ref[0]


def _rel_mlp(x2, wi_s, bi_s, wo_s, bo_s, off):
    wsel = lambda i: (jnp.minimum((i + off) // BPR, 2), 0, 0)
    return pl.pallas_call(
        _mlp_body,
        grid=(NBLK_H,),
        in_specs=[
            pl.BlockSpec((BLK, 2 * D), lambda i: (i, 0)),
            pl.BlockSpec((1, 2 * D, 2 * D), wsel),
            pl.BlockSpec((1, 1, 2 * D), wsel),
            pl.BlockSpec((1, 2 * D, 2 * D), wsel),
            pl.BlockSpec((1, 1, 2 * D), wsel),
        ],
        out_specs=pl.BlockSpec((BLK, 2 * D), lambda i: (i, 0)),
        out_shape=jax.ShapeDtypeStruct((H_ROWS // 2, 2 * D), jnp.float32),
    )(x2, wi_s, bi_s, wo_s, bo_s)


def _upd_body(pa0, pa1, pb0, pb1, h_ref, wt_ref, wb_ref, bi_ref, wo_ref, bo_ref,
              o_ref):
    sm = pa0[0] + pa1[0] + pb0[0] + pb1[0]
    h = h_ref[...]
    z = _mish(_dot(sm, wt_ref[...]) + _dot(h, wb_ref[...]) + bi_ref[...])
    o_ref[...] = h + _dot(z, wo_ref[...]) + bo_ref[...]


def _update(pa, pb, h, wt, wb, bi, wo, bo):
    pspec = lambda ci: pl.BlockSpec((1, UBLK, D), lambda i, ci=ci: (ci, i, 0))
    return pl.pallas_call(
        _upd_body,
        grid=(N // UBLK,),
        in_specs=[
            pspec(0), pspec(1), pspec(0), pspec(1),
            pl.BlockSpec((UBLK, D), lambda i: (i, 0)),
            pl.BlockSpec((D, 2 * D), lambda i: (0, 0)),
            pl.BlockSpec((D, 2 * D), lambda i: (0, 0)),
            pl.BlockSpec((1, 2 * D), lambda i: (0, 0)),
            pl.BlockSpec((2 * D, D), lambda i: (0, 0)),
            pl.BlockSpec((1, D), lambda i: (0, 0)),
        ],
        out_specs=pl.BlockSpec((UBLK, D), lambda i: (i, 0)),
        out_shape=jax.ShapeDtypeStruct((N, D), jnp.float32),
    )(pa, pa, pb, pb, h, wt, wb, bi, wo, bo)


def _blockdiag(w):
    z = jnp.zeros((2 * D, 2 * D), jnp.float32)
    return z.at[:D, :D].set(w).at[D:, D:].set(w)


def kernel(node_embeddings, atoms_adj, atoms_goal_adj, atoms_label,
           Wi_adj, bi_adj, Wo_adj, bo_adj,
           Wi_goal_adj, bi_goal_adj, Wo_goal_adj, bo_goal_adj,
           Wi_label, bi_label, Wo_label, bo_label,
           Wi_upd, bi_upd, Wo_upd, bo_upd):
    idx = jnp.concatenate([atoms_adj, atoms_goal_adj, atoms_label]).astype(jnp.int32)
    gidx = jnp.concatenate(
        [idx, jnp.zeros((B_PAD - B,), jnp.int32)]).reshape(2, NW, CPC, CHUNK)
    sidx = jnp.concatenate(
        [idx, jnp.full((B_PAD - B,), N, jnp.int32)]).reshape(2, NW, CPC, CHUNK)
    zeros_acc = jnp.zeros((ACC_ROWS, D), jnp.float32)

    wi_s = jnp.stack([Wi_adj, Wi_goal_adj, _blockdiag(Wi_label)])
    wo_s = jnp.stack([Wo_adj, Wo_goal_adj, _blockdiag(Wo_label)])
    bi_s = jnp.stack([bi_adj, bi_goal_adj,
                      jnp.concatenate([bi_label, bi_label])]).reshape(3, 1, 2 * D)
    bo_s = jnp.stack([bo_adj, bo_goal_adj,
                      jnp.concatenate([bo_label, bo_label])]).reshape(3, 1, 2 * D)

    wt = Wi_upd[:D]
    wb = Wi_upd[D:]
    bi_u = bi_upd.reshape(1, 2 * D)
    bo_u = bo_upd.reshape(1, D)

    h = node_embeddings
    for _ in range(2):
        xa = _gather_k(h, gidx[0])
        ya = _rel_mlp(xa.reshape(H_ROWS // 2, 2 * D), wi_s, bi_s, wo_s, bo_s, 0)
        xb = _gather_k(h, gidx[1])
        yb = _rel_mlp(xb.reshape(H_ROWS // 2, 2 * D), wi_s, bi_s, wo_s, bo_s,
                      NBLK_H)
        pa = _scatter_k(ya.reshape(H_ROWS, D), sidx[0], zeros_acc)
        pb = _scatter_k(yb.reshape(H_ROWS, D), sidx[1], zeros_acc)
        h = _update(pa, pb, h, wt, wb, bi_u, Wo_upd, bo_u)
    return h
